# baseline (speedup 1.0000x reference)
import functools

import jax
import jax.numpy as jnp
from jax import lax
from jax.experimental import pallas as pl
from jax.experimental.pallas import tpu as pltpu
from jax.experimental.pallas import tpu_sc as plsc

N_NODES = 10000
N_EDGES = 320000
D = 128

NC = 2
NS = 16
NW = NC * NS

CH = 128
N_CHUNKS = N_EDGES // CH
NB = 78
ACC_N = 10112
RPT = ACC_N // NS
IDX_WIN = 88
DST_PAD = 2504


def _sc_segment_sum(edge_features, dst_chunks):
    mesh = plsc.VectorSubcoreMesh(core_axis_name="c", subcore_axis_name="s")

    @functools.partial(
        pl.kernel,
        out_type=jax.ShapeDtypeStruct((NC, ACC_N, D), jnp.float32),
        mesh=mesh,
        scratch_types=[
            pltpu.VMEM((IDX_WIN, CH), jnp.int32),
            pltpu.VMEM((CH, D), jnp.float32),
            pltpu.VMEM((CH, D), jnp.float32),
            pltpu.VMEM_SHARED((ACC_N, D), jnp.float32),
            pltpu.SemaphoreType.DMA,
            pltpu.SemaphoreType.DMA,
        ],
    )
    def body(edge_hbm, idx_hbm, out_hbm, idx_v, ebufA, ebufB, acc,
             semA, semB):
        c = lax.axis_index("c")
        s = lax.axis_index("s")
        wid = c * NS + s

        cnt = jnp.int32(NB) + jnp.where(wid < 4, jnp.int32(1), jnp.int32(0))
        start = wid * jnp.int32(NB) + jnp.minimum(wid, jnp.int32(4))
        ws8 = pl.multiple_of(start - lax.rem(start, jnp.int32(8)), 8)
        off = start - ws8

        def load(slot, buf, sem):
            return pltpu.async_copy(
                edge_hbm.at[pl.ds((start + slot) * CH, CH)], buf, sem)

        def wait_load(buf, sem):
            pltpu.make_async_copy(edge_hbm.at[pl.ds(0, CH)], buf, sem).wait()

        def scatter(buf, slot):
            pltpu.sync_copy(buf, acc.at[idx_v.at[off + slot]], add=True)

        pltpu.async_copy(idx_hbm.at[pl.ds(ws8, IDX_WIN)], idx_v, semB)
        load(jnp.int32(0), ebufA, semA)

        def zrow(r, _):
            for q in range(D // 16):
                ebufB[r, pl.ds(q * 16, 16)] = jnp.zeros((16,), jnp.float32)
            return 0
        lax.fori_loop(0, CH, zrow, 0)
        base_r = s * RPT
        nfull = RPT // CH
        for k in range(nfull):
            pltpu.sync_copy(ebufB, acc.at[pl.ds(base_r + k * CH, CH)])
        rem = RPT - nfull * CH
        if rem:
            pltpu.sync_copy(ebufB.at[pl.ds(0, rem)],
                            acc.at[pl.ds(base_r + nfull * CH, rem)])
        pltpu.make_async_copy(idx_hbm.at[pl.ds(0, IDX_WIN)], idx_v,
                              semB).wait()
        plsc.subcore_barrier()

        def pipe(i, _):
            load(2 * i + 1, ebufB, semB)
            wait_load(ebufA, semA)
            scatter(ebufA, 2 * i)
            load(jnp.minimum(2 * i + 2, jnp.int32(NB - 1)), ebufA, semA)
            wait_load(ebufB, semB)
            scatter(ebufB, 2 * i + 1)
            return 0
        lax.fori_loop(0, NB // 2, pipe, 0)
        wait_load(ebufA, semA)

        @pl.when(cnt > NB)
        def _():
            pltpu.sync_copy(edge_hbm.at[pl.ds((start + NB) * CH, CH)], ebufA)
            scatter(ebufA, jnp.int32(NB))

        plsc.subcore_barrier()
        pltpu.sync_copy(acc.at[pl.ds(base_r, RPT)],
                        out_hbm.at[c, pl.ds(base_r, RPT)])

    return body(edge_features, dst_chunks)


def _tc_mlp(partials, node_features, W1a, W1b, b1, W2, b2, W3, b3, gamma, beta):
    BN = 2000
    grid = N_NODES // BN

    f32 = jnp.float32

    def body(p_ref, nf_ref, w1a_ref, w1b_ref, b1_ref, w2_ref, b2_ref,
             w3_ref, b3_ref, g_ref, bt_ref, out_ref):
        hd = p_ref[0] + p_ref[1]
        nf = nf_ref[...]
        h = (jnp.dot(hd.astype(jnp.bfloat16), w1a_ref[...],
                     preferred_element_type=f32)
             + jnp.dot(nf.astype(jnp.bfloat16), w1b_ref[...],
                       preferred_element_type=f32)
             + b1_ref[...])
        h = jnp.maximum(h, 0.0)
        h = jnp.maximum(
            jnp.dot(h.astype(jnp.bfloat16), w2_ref[...],
                    preferred_element_type=f32) + b2_ref[...], 0.0)
        h = (jnp.dot(h.astype(jnp.bfloat16), w3_ref[...],
                     preferred_element_type=f32) + b3_ref[...])
        mean = jnp.mean(h, axis=-1, keepdims=True)
        cent = h - mean
        var = jnp.mean(cent * cent, axis=-1, keepdims=True)
        h = cent * lax.rsqrt(var + 1e-5) * g_ref[...] + bt_ref[...]
        out_ref[...] = h + nf

    full = lambda shape: pl.BlockSpec(shape, lambda i: (0,) * len(shape))
    return pl.pallas_call(
        body,
        grid=(grid,),
        in_specs=[
            pl.BlockSpec((NC, BN, D), lambda i: (0, i, 0)),
            pl.BlockSpec((BN, D), lambda i: (i, 0)),
            full((D, D)), full((D, D)), full((1, D)),
            full((D, D)), full((1, D)),
            full((D, D)), full((1, D)),
            full((1, D)), full((1, D)),
        ],
        out_specs=pl.BlockSpec((BN, D), lambda i: (i, 0)),
        out_shape=jax.ShapeDtypeStruct((N_NODES, D), jnp.float32),
    )(partials, node_features, W1a, W1b, b1, W2, b2, W3, b3, gamma, beta)


def kernel(node_features, edge_features, edge_index, W1, b1, W2, b2, W3, b3,
           gamma, beta):
    dst = edge_index[1].astype(jnp.int32).reshape(N_CHUNKS, CH)
    dst_chunks = jnp.concatenate(
        [dst, jnp.zeros((DST_PAD - N_CHUNKS, CH), jnp.int32)], axis=0)
    partials = _sc_segment_sum(edge_features, dst_chunks)

    bf = jnp.bfloat16
    W1a = W1[:D].astype(bf)
    W1b = W1[D:].astype(bf)
    r1 = lambda v: v.reshape(1, D)
    return _tc_mlp(partials, node_features, W1a, W1b, r1(b1),
                   W2.astype(bf), r1(b2), W3.astype(bf), r1(b3),
                   r1(gamma), r1(beta))


# speedup vs baseline: 1.0160x; 1.0160x over previous
import functools

import jax
import jax.numpy as jnp
from jax import lax
from jax.experimental import pallas as pl
from jax.experimental.pallas import tpu as pltpu
from jax.experimental.pallas import tpu_sc as plsc

N_NODES = 10000
N_EDGES = 320000
D = 128

NC = 2
NS = 16
NW = NC * NS

CH = 128
N_CHUNKS = N_EDGES // CH
NB = 78
ACC_N = 10112
RPT = ACC_N // NS
IDX_WIN = 88
DST_PAD = 2504


def _sc_segment_sum(edge_features, dst_chunks):
    mesh = plsc.VectorSubcoreMesh(core_axis_name="c", subcore_axis_name="s")

    @functools.partial(
        pl.kernel,
        out_type=jax.ShapeDtypeStruct((NC, ACC_N, D), jnp.float32),
        mesh=mesh,
        scratch_types=[
            pltpu.VMEM((IDX_WIN, CH), jnp.int32),
            pltpu.VMEM((CH, D), jnp.float32),
            pltpu.VMEM((CH, D), jnp.float32),
            pltpu.VMEM_SHARED((ACC_N, D), jnp.float32),
            pltpu.SemaphoreType.DMA,
            pltpu.SemaphoreType.DMA,
        ],
    )
    def body(edge_hbm, idx_hbm, out_hbm, idx_v, ebufA, ebufB, acc,
             semA, semB):
        c = lax.axis_index("c")
        s = lax.axis_index("s")
        wid = c * NS + s

        cnt = jnp.int32(NB) + jnp.where(wid < 4, jnp.int32(1), jnp.int32(0))
        start = wid * jnp.int32(NB) + jnp.minimum(wid, jnp.int32(4))
        ws8 = pl.multiple_of(start - lax.rem(start, jnp.int32(8)), 8)
        off = start - ws8

        def load(slot, buf, sem):
            return pltpu.async_copy(
                edge_hbm.at[pl.ds((start + slot) * CH, CH)], buf, sem)

        def wait_load(buf, sem):
            pltpu.make_async_copy(edge_hbm.at[pl.ds(0, CH)], buf, sem).wait()

        def scatter(buf, slot):
            pltpu.sync_copy(buf, acc.at[idx_v.at[off + slot]], add=True)

        pltpu.async_copy(idx_hbm.at[pl.ds(ws8, IDX_WIN)], idx_v, semB)
        load(jnp.int32(0), ebufA, semA)

        def zrow(r, _):
            for q in range(D // 16):
                ebufB[r, pl.ds(q * 16, 16)] = jnp.zeros((16,), jnp.float32)
            return 0
        lax.fori_loop(0, CH, zrow, 0)
        base_r = s * RPT
        nfull = RPT // CH
        for k in range(nfull):
            pltpu.async_copy(ebufB, acc.at[pl.ds(base_r + k * CH, CH)], semB)
        rem = RPT - nfull * CH
        if rem:
            pltpu.async_copy(ebufB.at[pl.ds(0, rem)],
                             acc.at[pl.ds(base_r + nfull * CH, rem)], semB)
        for k in range(nfull):
            pltpu.make_async_copy(
                ebufB, acc.at[pl.ds(base_r + k * CH, CH)], semB).wait()
        if rem:
            pltpu.make_async_copy(
                ebufB.at[pl.ds(0, rem)],
                acc.at[pl.ds(base_r + nfull * CH, rem)], semB).wait()
        pltpu.make_async_copy(idx_hbm.at[pl.ds(0, IDX_WIN)], idx_v,
                              semB).wait()
        plsc.subcore_barrier()

        def pipe(i, _):
            load(2 * i + 1, ebufB, semB)
            wait_load(ebufA, semA)
            scatter(ebufA, 2 * i)
            load(jnp.minimum(2 * i + 2, cnt - 1), ebufA, semA)
            wait_load(ebufB, semB)
            scatter(ebufB, 2 * i + 1)
            return 0
        lax.fori_loop(0, NB // 2, pipe, 0)
        wait_load(ebufA, semA)
        @pl.when(cnt > NB)
        def _():
            scatter(ebufA, jnp.int32(NB))

        plsc.subcore_barrier()
        pltpu.sync_copy(acc.at[pl.ds(base_r, RPT)],
                        out_hbm.at[c, pl.ds(base_r, RPT)])

    return body(edge_features, dst_chunks)


def _tc_mlp(partials, node_features, W1a, W1b, b1, W2, b2, W3, b3, gamma, beta):
    BN = 2000
    grid = N_NODES // BN

    def body(p_ref, nf_ref, w1a_ref, w1b_ref, b1_ref, w2_ref, b2_ref,
             w3_ref, b3_ref, g_ref, bt_ref, out_ref):
        hd = p_ref[0] + p_ref[1]
        nf = nf_ref[...]
        h = (jnp.dot(hd, w1a_ref[...], preferred_element_type=jnp.float32)
             + jnp.dot(nf, w1b_ref[...], preferred_element_type=jnp.float32)
             + b1_ref[...])
        h = jnp.maximum(h, 0.0)
        h = jnp.maximum(
            jnp.dot(h, w2_ref[...], preferred_element_type=jnp.float32)
            + b2_ref[...], 0.0)
        h = (jnp.dot(h, w3_ref[...], preferred_element_type=jnp.float32)
             + b3_ref[...])
        mean = jnp.mean(h, axis=-1, keepdims=True)
        cent = h - mean
        var = jnp.mean(cent * cent, axis=-1, keepdims=True)
        h = cent * lax.rsqrt(var + 1e-5) * g_ref[...] + bt_ref[...]
        out_ref[...] = h + nf

    full = lambda shape: pl.BlockSpec(shape, lambda i: (0,) * len(shape))
    return pl.pallas_call(
        body,
        grid=(grid,),
        in_specs=[
            pl.BlockSpec((NC, BN, D), lambda i: (0, i, 0)),
            pl.BlockSpec((BN, D), lambda i: (i, 0)),
            full((D, D)), full((D, D)), full((1, D)),
            full((D, D)), full((1, D)),
            full((D, D)), full((1, D)),
            full((1, D)), full((1, D)),
        ],
        out_specs=pl.BlockSpec((BN, D), lambda i: (i, 0)),
        out_shape=jax.ShapeDtypeStruct((N_NODES, D), jnp.float32),
    )(partials, node_features, W1a, W1b, b1, W2, b2, W3, b3, gamma, beta)


def kernel(node_features, edge_features, edge_index, W1, b1, W2, b2, W3, b3,
           gamma, beta):
    dst = edge_index[1].astype(jnp.int32).reshape(N_CHUNKS, CH)
    dst_chunks = jnp.concatenate(
        [dst, jnp.zeros((DST_PAD - N_CHUNKS, CH), jnp.int32)], axis=0)
    partials = _sc_segment_sum(edge_features, dst_chunks)

    W1a = W1[:D]
    W1b = W1[D:]
    r1 = lambda v: v.reshape(1, D)
    return _tc_mlp(partials, node_features, W1a, W1b, r1(b1), W2, r1(b2),
                   W3, r1(b3), r1(gamma), r1(beta))
